# baseline (speedup 1.0000x reference)
import jax
import jax.numpy as jnp
from jax.experimental import pallas as pl
from jax.experimental.pallas import tpu as pltpu

N = 1024
F = 128
OUT = 128
K = 9
BI = 8
CHUNK = 16
NEGB = float(-3e38)


def _body(adj_ref, fea_ref, dsel_ref, w_ref, b_ref, out_ref, topk_ref,
          pan_ref):
    a8t = adj_ref[0]
    pan_ref[:] = jax.lax.dot_general(
        a8t, dsel_ref[:], (((0,), (0,)), ((), ())),
        preferred_element_type=jnp.float32).astype(jnp.bfloat16)
    for d in range(BI):

        def s1_body(c, RR, d=d):
            RR = list(RR)
            for h in range(2):
                v = (pan_ref[pl.ds((h * (N // CHUNK // 2) + c) * CHUNK,
                                   CHUNK), d * F : (d + 1) * F]
                     * fea_ref[pl.ds((h * (N // CHUNK // 2) + c) * CHUNK,
                                     CHUNK), :])
                for t in range(K):
                    hi = jnp.maximum(RR[h * K + t], v)
                    v = jnp.minimum(RR[h * K + t], v)
                    RR[h * K + t] = hi
            return tuple(RR)

        R0 = tuple(jnp.full((CHUNK, F), NEGB, dtype=jnp.bfloat16)
                   for _ in range(2 * K))
        RR = jax.lax.fori_loop(0, N // CHUNK // 2, s1_body, R0, unroll=8)
        pos = [jnp.maximum(RR[t], RR[K + 8 - t]) for t in range(K)] + [None] * 7
        for m in (8, 4, 2, 1):
            for i in range(16):
                if (i % (2 * m)) < m and i + m < 16:
                    a, b = pos[i], pos[i + m]
                    if a is None and b is None:
                        continue
                    if a is None:
                        pos[i + m] = b
                        continue
                    if b is None:
                        pos[i], pos[i + m] = None, a
                        continue
                    pos[i], pos[i + m] = jnp.maximum(a, b), jnp.minimum(a, b)
        R = [p for p in pos if p is not None]
        c32 = jnp.concatenate([r.astype(jnp.float32) for r in R], axis=0)
        cand = jnp.concatenate(
            [jnp.maximum(c32[t * CHUNK : t * CHUNK + 8, :],
                         c32[(8 - t) * CHUNK + 8 : (8 - t) * CHUNK + 16, :])
             for t in range(K)], axis=0)
        ids = jax.lax.broadcasted_iota(jnp.int32, (K * 8, F), 0)
        keys = jax.lax.bitcast_convert_type(
            jax.lax.bitcast_convert_type(cand, jnp.int32) | ids, jnp.float32)

        def s2_body(t, m, d=d):
            val = jax.lax.bitcast_convert_type(
                jax.lax.bitcast_convert_type(m, jnp.int32) & (~0xFF),
                jnp.float32)
            topk_ref[pl.ds(t * BI + d, 1), :] = val
            return jnp.max(jnp.where(keys < m, keys, NEGB), axis=0,
                           keepdims=True)

        m0 = jnp.max(keys, axis=0, keepdims=True)
        jax.lax.fori_loop(0, K, s2_body, m0)

    acc = jnp.zeros((BI, OUT), dtype=jnp.float32)
    for t in range(K):
        acc += jnp.dot(topk_ref[t * BI : (t + 1) * BI, :], w_ref[t],
                       preferred_element_type=jnp.float32)
    out_ref[:] = jnp.maximum(acc + b_ref[:], 0.0)


@jax.jit
def kernel(inputs, support, W, b):
    b2 = b.reshape(1, OUT)
    fea = inputs.astype(jnp.bfloat16)
    support3 = support.T.astype(jnp.bfloat16).reshape(N // BI, BI, N)
    dsel = (jnp.arange(BI, dtype=jnp.int32)[:, None]
            == (jnp.arange(BI * F, dtype=jnp.int32) // F)[None, :]
            ).astype(jnp.bfloat16)
    grid = (N // BI,)
    return pl.pallas_call(
        _body,
        grid=grid,
        in_specs=[
            pl.BlockSpec((1, BI, N), lambda ib: (ib, 0, 0)),
            pl.BlockSpec((N, F), lambda ib: (0, 0)),
            pl.BlockSpec((BI, BI * F), lambda ib: (0, 0)),
            pl.BlockSpec((K, F, OUT), lambda ib: (0, 0, 0)),
            pl.BlockSpec((1, OUT), lambda ib: (0, 0)),
        ],
        out_specs=pl.BlockSpec((BI, OUT), lambda ib: (ib, 0)),
        out_shape=jax.ShapeDtypeStruct((N, OUT), jnp.float32),
        scratch_shapes=[
            pltpu.VMEM((K * BI, F), jnp.float32),
            pltpu.VMEM((N, BI * F), jnp.bfloat16),
        ],
    )(support3, fea, dsel, W, b2)
